# baseline (speedup 1.0000x reference)
import functools

import jax
import jax.numpy as jnp
from jax import lax
from jax.experimental import pallas as pl
from jax.experimental.pallas import tpu as pltpu
from jax.experimental.pallas import tpu_sc as plsc

N = 10000
E = 320000
H = 32
EDGE_NF = 16
N_LAYERS = 4

NC = 2
NS = 16
NW = NC * NS

NSHARD = 1
ESH = E // NSHARD

GC = 1000
_STREAMS = [(i * 128, 128) for i in range(7)] + [(896, 104)]

NT = N // NS


@functools.lru_cache(maxsize=None)
def _make_sc_gather(ne):
    ew = ne // NW
    n_chunk = ew // GC
    assert ew % GC == 0
    mesh = plsc.VectorSubcoreMesh(core_axis_name="c", subcore_axis_name="s",
                                  num_cores=NC, num_subcores=NS)

    def body(h_hbm, xp_hbm, row_hbm, col_hbm,
             hr_out, hc_out, xr_out, xc_out,
             idx_r, idx_c, bhr, bhc, bxr, bxc, sem):
        w = lax.axis_index("s") * NC + lax.axis_index("c")

        def chunk(k, _):
            base = w * ew + k * GC
            pltpu.sync_copy(row_hbm.at[pl.ds(base, GC)], idx_r)
            pltpu.sync_copy(col_hbm.at[pl.ds(base, GC)], idx_c)
            copies = []
            for off, ln in _STREAMS:
                copies.append(pltpu.async_copy(
                    h_hbm.at[idx_r.at[pl.ds(off, ln)]],
                    bhr.at[pl.ds(off, ln)], sem))
                copies.append(pltpu.async_copy(
                    h_hbm.at[idx_c.at[pl.ds(off, ln)]],
                    bhc.at[pl.ds(off, ln)], sem))
                copies.append(pltpu.async_copy(
                    xp_hbm.at[idx_r.at[pl.ds(off, ln)]],
                    bxr.at[pl.ds(off, ln)], sem))
                copies.append(pltpu.async_copy(
                    xp_hbm.at[idx_c.at[pl.ds(off, ln)]],
                    bxc.at[pl.ds(off, ln)], sem))
            for cp in copies:
                cp.wait()
            pltpu.sync_copy(bhr, hr_out.at[pl.ds(base, GC)])
            pltpu.sync_copy(bhc, hc_out.at[pl.ds(base, GC)])
            pltpu.sync_copy(bxr, xr_out.at[pl.ds(base, GC)])
            pltpu.sync_copy(bxc, xc_out.at[pl.ds(base, GC)])
            return ()

        lax.fori_loop(0, n_chunk, chunk, (), unroll=False)

    return pl.kernel(
        body,
        out_type=(
            jax.ShapeDtypeStruct((ne, H), jnp.float32),
            jax.ShapeDtypeStruct((ne, H), jnp.float32),
            jax.ShapeDtypeStruct((ne, 8), jnp.float32),
            jax.ShapeDtypeStruct((ne, 8), jnp.float32),
        ),
        mesh=mesh,
        scratch_types=dict(
            idx_r=pltpu.VMEM((GC,), jnp.int32),
            idx_c=pltpu.VMEM((GC,), jnp.int32),
            bhr=pltpu.VMEM((GC, H), jnp.float32),
            bhc=pltpu.VMEM((GC, H), jnp.float32),
            bxr=pltpu.VMEM((GC, 8), jnp.float32),
            bxc=pltpu.VMEM((GC, 8), jnp.float32),
            sem=pltpu.SemaphoreType.DMA,
        ),
        compiler_params=pltpu.CompilerParams(use_tc_tiling_on_sc=False),
    )


@functools.lru_cache(maxsize=None)
def _make_sc_scatter(ne):
    ns = ne // 128
    spw = ns // NW
    nfull = spw // 8
    tail = spw % 8
    nleft = ns - NW * spw
    mesh = plsc.VectorSubcoreMesh(core_axis_name="c", subcore_axis_name="s",
                                  num_cores=NC, num_subcores=NS)

    def body(medge, tedge, row2d, zm_hbm, zt_hbm,
             mpart, tpart,
             idx2, mbuf, tbuf, acc_m, acc_t):
        c = lax.axis_index("c")
        s = lax.axis_index("s")
        w = s * NC + c

        pltpu.sync_copy(zm_hbm.at[pl.ds(s * NT, NT)],
                        acc_m.at[pl.ds(s * NT, NT)])
        pltpu.sync_copy(zt_hbm.at[pl.ds(s * NT, NT)],
                        acc_t.at[pl.ds(s * NT, NT)])
        plsc.subcore_barrier()

        def batch(sb, nst):
            nedge = nst * 128
            pltpu.sync_copy(row2d.at[pl.ds(sb, nst)], idx2.at[pl.ds(0, nst)])
            pltpu.sync_copy(medge.at[pl.ds(sb * 128, nedge)],
                            mbuf.at[pl.ds(0, nedge)])
            pltpu.sync_copy(tedge.at[pl.ds(sb * 128, nedge)],
                            tbuf.at[pl.ds(0, nedge)])
            for j in range(nst):
                pltpu.sync_copy(mbuf.at[pl.ds(j * 128, 128)],
                                acc_m.at[idx2.at[j]], add=True)
                pltpu.sync_copy(tbuf.at[pl.ds(j * 128, 128)],
                                acc_t.at[idx2.at[j]], add=True)

        def full_batch(k, _):
            batch(w * spw + k * 8, 8)
            return ()

        lax.fori_loop(0, nfull, full_batch, (), unroll=False)
        if tail:
            batch(w * spw + nfull * 8, tail)
        if nleft:
            @pl.when(w < nleft)
            def _():
                batch(NW * spw + w, 1)

        plsc.subcore_barrier()
        pltpu.sync_copy(acc_m.at[pl.ds(s * NT, NT)],
                        mpart.at[c, pl.ds(s * NT, NT)])
        pltpu.sync_copy(acc_t.at[pl.ds(s * NT, NT)],
                        tpart.at[c, pl.ds(s * NT, NT)])

    return pl.kernel(
        body,
        out_type=(
            jax.ShapeDtypeStruct((NC, N, H), jnp.float32),
            jax.ShapeDtypeStruct((NC, N, 8), jnp.float32),
        ),
        mesh=mesh,
        scratch_types=dict(
            idx2=pltpu.VMEM((8, 128), jnp.int32),
            mbuf=pltpu.VMEM((1024, H), jnp.float32),
            tbuf=pltpu.VMEM((1024, 8), jnp.float32),
            acc_m=pltpu.VMEM_SHARED((N, H), jnp.float32),
            acc_t=pltpu.VMEM_SHARED((N, 8), jnp.float32),
        ),
        compiler_params=pltpu.CompilerParams(use_tc_tiling_on_sc=False),
    )


BE = 2000


def _edge_body(hr, hc, xr, xc, ea, We1, be1, We2, be2, Wc1, bc1, Wc2,
               m_out, t_out):
    silu = jax.nn.silu
    xd = xr[...] - xc[...]
    radial = jnp.sum(xd * xd, axis=1, keepdims=True)
    w1 = We1[...]
    rea = jnp.concatenate([radial, ea[...]], axis=1)
    t1 = (jnp.dot(hr[...], w1[0:H], preferred_element_type=jnp.float32)
          + jnp.dot(hc[...], w1[H:2 * H], preferred_element_type=jnp.float32)
          + jnp.dot(rea, w1[2 * H:], preferred_element_type=jnp.float32)
          + be1[...])
    m1 = silu(t1)
    m = silu(jnp.dot(m1, We2[...], preferred_element_type=jnp.float32) + be2[...])
    q = silu(jnp.dot(m, Wc1[...], preferred_element_type=jnp.float32) + bc1[...])
    p = jnp.dot(q, Wc2[...], preferred_element_type=jnp.float32)
    m_out[...] = m
    colid = lax.broadcasted_iota(jnp.int32, (BE, 8), 1)
    t_out[...] = jnp.where(colid == 3, 1.0, xd * p)


def _edge_call(hr, hc, xr, xc, ea, We1, be1, We2, be2, Wc1, bc1, Wc2):
    ne = hr.shape[0]
    bspec = lambda shape: pl.BlockSpec(shape, lambda i: (i, 0))
    wspec = lambda shape: pl.BlockSpec(shape, lambda i: (0, 0))
    vspec = pl.BlockSpec((H,), lambda i: (0,))
    return pl.pallas_call(
        _edge_body,
        grid=(ne // BE,),
        in_specs=[
            bspec((BE, H)), bspec((BE, H)), bspec((BE, 8)), bspec((BE, 8)),
            bspec((BE, EDGE_NF)),
            wspec((2 * H + 1 + EDGE_NF, H)), vspec,
            wspec((H, H)), vspec,
            wspec((H, H)), vspec,
            wspec((H, 1)),
        ],
        out_specs=[bspec((BE, H)), bspec((BE, 8))],
        out_shape=[
            jax.ShapeDtypeStruct((ne, H), jnp.float32),
            jax.ShapeDtypeStruct((ne, 8), jnp.float32),
        ],
    )(hr, hc, xr, xc, ea, We1, be1, We2, be2, Wc1, bc1, Wc2)


BN = 2000


def _node_body(h, m0, m1, m2, m3, t0, t1, t2, t3, vel, x,
               Wv1, bv1, Wv2, bv2, Wn1, bn1, Wn2, bn2,
               x_out, v_out, h_out):
    silu = jax.nn.silu
    hv = h[...]
    n_agg = (m0[...] + m1[...]) + (m2[...] + m3[...])
    tsum = (t0[...] + t1[...]) + (t2[...] + t3[...])
    counts = jnp.maximum(tsum[:, 3:4], 1.0)
    colid = lax.broadcasted_iota(jnp.int32, (BN, 8), 1)
    aggp = jnp.where(colid < 3, tsum, 0.0) / counts
    scale = (jnp.dot(silu(jnp.dot(hv, Wv1[...],
                                  preferred_element_type=jnp.float32) + bv1[...]),
                     Wv2[...], preferred_element_type=jnp.float32) + bv2[...])
    v_new = scale * vel[...] + aggp
    x_out[...] = x[...] + v_new
    v_out[...] = v_new
    cat = jnp.concatenate([hv, n_agg], axis=1)
    hmid = silu(jnp.dot(cat, Wn1[...], preferred_element_type=jnp.float32)
                + bn1[...])
    h_out[...] = hv + jnp.dot(hmid, Wn2[...],
                              preferred_element_type=jnp.float32) + bn2[...]


def _node_call(h, ms, ts, vel, x, Wv1, bv1, Wv2, bv2, Wn1, bn1, Wn2, bn2):
    bspec = lambda shape: pl.BlockSpec(shape, lambda i: (i, 0))
    wspec = lambda shape: pl.BlockSpec(shape, lambda i: (0, 0))
    vspec = lambda n: pl.BlockSpec((n,), lambda i: (0,))
    return pl.pallas_call(
        _node_body,
        grid=(N // BN,),
        in_specs=[
            bspec((BN, H)),
            bspec((BN, H)), bspec((BN, H)), bspec((BN, H)), bspec((BN, H)),
            bspec((BN, 8)), bspec((BN, 8)), bspec((BN, 8)), bspec((BN, 8)),
            bspec((BN, 8)), bspec((BN, 8)),
            wspec((H, H)), vspec(H), wspec((H, 1)), vspec(1),
            wspec((2 * H, H)), vspec(H), wspec((H, H)), vspec(H),
        ],
        out_specs=[bspec((BN, 8)), bspec((BN, 8)), bspec((BN, H))],
        out_shape=[
            jax.ShapeDtypeStruct((N, 8), jnp.float32),
            jax.ShapeDtypeStruct((N, 8), jnp.float32),
            jax.ShapeDtypeStruct((N, H), jnp.float32),
        ],
    )(h, *ms, *ts, vel, x, Wv1, bv1, Wv2, bv2, Wn1, bn1, Wn2, bn2)


def _emb_body(his, W, b, h_out):
    h_out[...] = jnp.dot(his[...], W[...],
                         preferred_element_type=jnp.float32) + b[...]


def _emb_call(his, W, b):
    return pl.pallas_call(
        _emb_body,
        out_shape=jax.ShapeDtypeStruct((N, H), jnp.float32),
    )(his, W, b)


def kernel(his, x, edges, v, edge_attr, W_emb, b_emb, We1, be1, We2, be2,
           Wc1, bc1, Wc2, Wv1, bv1, Wv2, bv2, Wn1, bn1, Wn2, bn2):
    rows = [edges[0, i * ESH:(i + 1) * ESH] for i in range(NSHARD)]
    cols = [edges[1, i * ESH:(i + 1) * ESH] for i in range(NSHARD)]
    row2ds = [r.reshape(ESH // 128, 128) for r in rows]
    eas = [edge_attr[i * ESH:(i + 1) * ESH] for i in range(NSHARD)]
    xp = jnp.pad(x, ((0, 0), (0, 5)))
    vp = jnp.pad(v, ((0, 0), (0, 5)))
    zm = jnp.zeros((N, H), jnp.float32)
    zt = jnp.zeros((N, 8), jnp.float32)

    gather = _make_sc_gather(ESH)
    scatter = _make_sc_scatter(ESH)

    h = _emb_call(his, W_emb, b_emb)
    for _ in range(N_LAYERS):
        gat = [gather(h, xp, rows[i], cols[i]) for i in range(NSHARD)]
        edg = [_edge_call(*gat[i], eas[i], We1, be1, We2, be2, Wc1, bc1, Wc2)
               for i in range(NSHARD)]
        par = [scatter(edg[i][0], edg[i][1], row2ds[i], zm, zt)
               for i in range(NSHARD)]
        ms = [p[0][c] for p in par for c in range(NC)]
        ts = [p[1][c] for p in par for c in range(NC)]
        if NSHARD == 1:
            ms = ms + [zm, zm]
            ts = ts + [zt, zt]
        xp, vp, h = _node_call(h, ms, ts, vp, xp,
                               Wv1, bv1, Wv2, bv2, Wn1, bn1, Wn2, bn2)
    return (xp[:, :3], h, vp[:, :3])


# speedup vs baseline: 1.1235x; 1.1235x over previous
import functools

import jax
import jax.numpy as jnp
from jax import lax
from jax.experimental import pallas as pl
from jax.experimental.pallas import tpu as pltpu
from jax.experimental.pallas import tpu_sc as plsc

N = 10000
E = 320000
H = 32
EDGE_NF = 16
N_LAYERS = 4

NC = 2
NS = 16
NW = NC * NS

NSHARD = 1
ESH = E // NSHARD

GC = 1000
_STREAMS = [(i * 128, 128) for i in range(7)] + [(896, 104)]

NT = N // NS


@functools.lru_cache(maxsize=None)
def _make_sc_gather(ne):
    ew = ne // NW
    n_chunk = ew // GC
    assert ew % GC == 0
    mesh = plsc.VectorSubcoreMesh(core_axis_name="c", subcore_axis_name="s",
                                  num_cores=NC, num_subcores=NS)

    def body(h_hbm, xp_hbm, row_hbm, col_hbm,
             hr_out, hc_out, xr_out, xc_out,
             idx_r, idx_c, bhr, bhc, bxr, bxc, sem):
        w = lax.axis_index("s") * NC + lax.axis_index("c")

        def chunk(k, _):
            base = w * ew + k * GC
            pltpu.sync_copy(row_hbm.at[pl.ds(base, GC)], idx_r)
            pltpu.sync_copy(col_hbm.at[pl.ds(base, GC)], idx_c)
            copies = []
            for off, ln in _STREAMS:
                copies.append(pltpu.async_copy(
                    h_hbm.at[idx_r.at[pl.ds(off, ln)]],
                    bhr.at[pl.ds(off, ln)], sem))
                copies.append(pltpu.async_copy(
                    h_hbm.at[idx_c.at[pl.ds(off, ln)]],
                    bhc.at[pl.ds(off, ln)], sem))
                copies.append(pltpu.async_copy(
                    xp_hbm.at[idx_r.at[pl.ds(off, ln)]],
                    bxr.at[pl.ds(off, ln)], sem))
                copies.append(pltpu.async_copy(
                    xp_hbm.at[idx_c.at[pl.ds(off, ln)]],
                    bxc.at[pl.ds(off, ln)], sem))
            for cp in copies:
                cp.wait()
            pltpu.sync_copy(bhr, hr_out.at[pl.ds(base, GC)])
            pltpu.sync_copy(bhc, hc_out.at[pl.ds(base, GC)])
            pltpu.sync_copy(bxr, xr_out.at[pl.ds(base, GC)])
            pltpu.sync_copy(bxc, xc_out.at[pl.ds(base, GC)])
            return ()

        lax.fori_loop(0, n_chunk, chunk, (), unroll=False)

    return pl.kernel(
        body,
        out_type=(
            jax.ShapeDtypeStruct((ne, H), jnp.float32),
            jax.ShapeDtypeStruct((ne, H), jnp.float32),
            jax.ShapeDtypeStruct((ne, 8), jnp.float32),
            jax.ShapeDtypeStruct((ne, 8), jnp.float32),
        ),
        mesh=mesh,
        scratch_types=dict(
            idx_r=pltpu.VMEM((GC,), jnp.int32),
            idx_c=pltpu.VMEM((GC,), jnp.int32),
            bhr=pltpu.VMEM((GC, H), jnp.float32),
            bhc=pltpu.VMEM((GC, H), jnp.float32),
            bxr=pltpu.VMEM((GC, 8), jnp.float32),
            bxc=pltpu.VMEM((GC, 8), jnp.float32),
            sem=pltpu.SemaphoreType.DMA,
        ),
        compiler_params=pltpu.CompilerParams(use_tc_tiling_on_sc=False),
    )


@functools.lru_cache(maxsize=None)
def _make_sc_scatter(ne):
    ns = ne // 128
    spw = ns // NW
    nfull = spw // 8
    tail = spw % 8
    nleft = ns - NW * spw
    mesh = plsc.VectorSubcoreMesh(core_axis_name="c", subcore_axis_name="s",
                                  num_cores=NC, num_subcores=NS)

    def body(medge, tedge, row2d, zm_hbm, zt_hbm,
             mpart, tpart,
             idx2, mbuf, tbuf, acc_m, acc_t):
        c = lax.axis_index("c")
        s = lax.axis_index("s")
        w = s * NC + c

        pltpu.sync_copy(zm_hbm.at[pl.ds(s * NT, NT)],
                        acc_m.at[pl.ds(s * NT, NT)])
        pltpu.sync_copy(zt_hbm.at[pl.ds(s * NT, NT)],
                        acc_t.at[pl.ds(s * NT, NT)])
        plsc.subcore_barrier()

        def batch(sb, nst):
            nedge = nst * 128
            pltpu.sync_copy(row2d.at[pl.ds(sb, nst)], idx2.at[pl.ds(0, nst)])
            pltpu.sync_copy(medge.at[pl.ds(sb * 128, nedge)],
                            mbuf.at[pl.ds(0, nedge)])
            pltpu.sync_copy(tedge.at[pl.ds(sb * 128, nedge)],
                            tbuf.at[pl.ds(0, nedge)])
            for j in range(nst):
                pltpu.sync_copy(mbuf.at[pl.ds(j * 128, 128)],
                                acc_m.at[idx2.at[j]], add=True)
                pltpu.sync_copy(tbuf.at[pl.ds(j * 128, 128)],
                                acc_t.at[idx2.at[j]], add=True)

        def full_batch(k, _):
            batch(w * spw + k * 8, 8)
            return ()

        lax.fori_loop(0, nfull, full_batch, (), unroll=False)
        if tail:
            batch(w * spw + nfull * 8, tail)
        if nleft:
            @pl.when(w < nleft)
            def _():
                batch(NW * spw + w, 1)

        plsc.subcore_barrier()
        pltpu.sync_copy(acc_m.at[pl.ds(s * NT, NT)],
                        mpart.at[c, pl.ds(s * NT, NT)])
        pltpu.sync_copy(acc_t.at[pl.ds(s * NT, NT)],
                        tpart.at[c, pl.ds(s * NT, NT)])

    return pl.kernel(
        body,
        out_type=(
            jax.ShapeDtypeStruct((NC, N, H), jnp.float32),
            jax.ShapeDtypeStruct((NC, N, 8), jnp.float32),
        ),
        mesh=mesh,
        scratch_types=dict(
            idx2=pltpu.VMEM((8, 128), jnp.int32),
            mbuf=pltpu.VMEM((1024, H), jnp.float32),
            tbuf=pltpu.VMEM((1024, 8), jnp.float32),
            acc_m=pltpu.VMEM_SHARED((N, H), jnp.float32),
            acc_t=pltpu.VMEM_SHARED((N, 8), jnp.float32),
        ),
        compiler_params=pltpu.CompilerParams(use_tc_tiling_on_sc=False),
    )


BE = 2000


def _edge_body(hr, hc, xr, xc, ea, We1, be1, We2, be2, Wc1, bc1, Wc2,
               m_out, t_out):
    silu = jax.nn.silu
    xd = xr[...] - xc[...]
    radial = jnp.sum(xd * xd, axis=1, keepdims=True)
    e_in = jnp.concatenate([hr[...], hc[...], radial, ea[...]], axis=1)
    t1 = jnp.dot(e_in, We1[...], preferred_element_type=jnp.float32) + be1[...]
    m1 = silu(t1)
    m = silu(jnp.dot(m1, We2[...], preferred_element_type=jnp.float32) + be2[...])
    q = silu(jnp.dot(m, Wc1[...], preferred_element_type=jnp.float32) + bc1[...])
    p = jnp.dot(q, Wc2[...], preferred_element_type=jnp.float32)
    m_out[...] = m
    colid = lax.broadcasted_iota(jnp.int32, (BE, 8), 1)
    t_out[...] = jnp.where(colid == 3, 1.0, xd * p)


def _edge_call(hr, hc, xr, xc, ea, We1, be1, We2, be2, Wc1, bc1, Wc2):
    ne = hr.shape[0]
    bspec = lambda shape: pl.BlockSpec(shape, lambda i: (i, 0))
    wspec = lambda shape: pl.BlockSpec(shape, lambda i: (0, 0))
    vspec = pl.BlockSpec((H,), lambda i: (0,))
    return pl.pallas_call(
        _edge_body,
        grid=(ne // BE,),
        in_specs=[
            bspec((BE, H)), bspec((BE, H)), bspec((BE, 8)), bspec((BE, 8)),
            bspec((BE, EDGE_NF)),
            wspec((2 * H + 1 + EDGE_NF, H)), vspec,
            wspec((H, H)), vspec,
            wspec((H, H)), vspec,
            wspec((H, 1)),
        ],
        out_specs=[bspec((BE, H)), bspec((BE, 8))],
        out_shape=[
            jax.ShapeDtypeStruct((ne, H), jnp.float32),
            jax.ShapeDtypeStruct((ne, 8), jnp.float32),
        ],
    )(hr, hc, xr, xc, ea, We1, be1, We2, be2, Wc1, bc1, Wc2)


BN = 2000


def _node_body(h, m0, m1, t0, t1, vel, x,
               Wv1, bv1, Wv2, bv2, Wn1, bn1, Wn2, bn2,
               x_out, v_out, h_out):
    silu = jax.nn.silu
    hv = h[...]
    n_agg = m0[...] + m1[...]
    tsum = t0[...] + t1[...]
    counts = jnp.maximum(tsum[:, 3:4], 1.0)
    colid = lax.broadcasted_iota(jnp.int32, (BN, 8), 1)
    aggp = jnp.where(colid < 3, tsum, 0.0) / counts
    scale = (jnp.dot(silu(jnp.dot(hv, Wv1[...],
                                  preferred_element_type=jnp.float32) + bv1[...]),
                     Wv2[...], preferred_element_type=jnp.float32) + bv2[...])
    v_new = scale * vel[...] + aggp
    x_out[...] = x[...] + v_new
    v_out[...] = v_new
    cat = jnp.concatenate([hv, n_agg], axis=1)
    hmid = silu(jnp.dot(cat, Wn1[...], preferred_element_type=jnp.float32)
                + bn1[...])
    h_out[...] = hv + jnp.dot(hmid, Wn2[...],
                              preferred_element_type=jnp.float32) + bn2[...]


def _node_call(h, ms, ts, vel, x, Wv1, bv1, Wv2, bv2, Wn1, bn1, Wn2, bn2):
    bspec = lambda shape: pl.BlockSpec(shape, lambda i: (i, 0))
    wspec = lambda shape: pl.BlockSpec(shape, lambda i: (0, 0))
    vspec = lambda n: pl.BlockSpec((n,), lambda i: (0,))
    return pl.pallas_call(
        _node_body,
        grid=(N // BN,),
        in_specs=[
            bspec((BN, H)),
            bspec((BN, H)), bspec((BN, H)),
            bspec((BN, 8)), bspec((BN, 8)),
            bspec((BN, 8)), bspec((BN, 8)),
            wspec((H, H)), vspec(H), wspec((H, 1)), vspec(1),
            wspec((2 * H, H)), vspec(H), wspec((H, H)), vspec(H),
        ],
        out_specs=[bspec((BN, 8)), bspec((BN, 8)), bspec((BN, H))],
        out_shape=[
            jax.ShapeDtypeStruct((N, 8), jnp.float32),
            jax.ShapeDtypeStruct((N, 8), jnp.float32),
            jax.ShapeDtypeStruct((N, H), jnp.float32),
        ],
    )(h, *ms, *ts, vel, x, Wv1, bv1, Wv2, bv2, Wn1, bn1, Wn2, bn2)


def _emb_body(his, W, b, h_out):
    h_out[...] = jnp.dot(his[...], W[...],
                         preferred_element_type=jnp.float32) + b[...]


def _emb_call(his, W, b):
    return pl.pallas_call(
        _emb_body,
        out_shape=jax.ShapeDtypeStruct((N, H), jnp.float32),
    )(his, W, b)


def kernel(his, x, edges, v, edge_attr, W_emb, b_emb, We1, be1, We2, be2,
           Wc1, bc1, Wc2, Wv1, bv1, Wv2, bv2, Wn1, bn1, Wn2, bn2):
    rows = [edges[0, i * ESH:(i + 1) * ESH] for i in range(NSHARD)]
    cols = [edges[1, i * ESH:(i + 1) * ESH] for i in range(NSHARD)]
    row2ds = [r.reshape(ESH // 128, 128) for r in rows]
    eas = [edge_attr[i * ESH:(i + 1) * ESH] for i in range(NSHARD)]
    xp = jnp.pad(x, ((0, 0), (0, 5)))
    vp = jnp.pad(v, ((0, 0), (0, 5)))
    zm = jnp.zeros((N, H), jnp.float32)
    zt = jnp.zeros((N, 8), jnp.float32)

    gather = _make_sc_gather(ESH)
    scatter = _make_sc_scatter(ESH)

    h = _emb_call(his, W_emb, b_emb)
    for _ in range(N_LAYERS):
        gat = [gather(h, xp, rows[i], cols[i]) for i in range(NSHARD)]
        edg = [_edge_call(*gat[i], eas[i], We1, be1, We2, be2, Wc1, bc1, Wc2)
               for i in range(NSHARD)]
        par = [scatter(edg[i][0], edg[i][1], row2ds[i], zm, zt)
               for i in range(NSHARD)]
        ms = [p[0][c] for p in par for c in range(NC)]
        ts = [p[1][c] for p in par for c in range(NC)]
        xp, vp, h = _node_call(h, ms, ts, vp, xp,
                               Wv1, bv1, Wv2, bv2, Wn1, bn1, Wn2, bn2)
    return (xp[:, :3], h, vp[:, :3])
